# baseline (speedup 1.0000x reference)
import functools

import jax
import jax.numpy as jnp
from jax import lax
from jax.experimental import pallas as pl
from jax.experimental.pallas import tpu as pltpu
from jax.experimental.pallas import tpu_sc as plsc

B, D_IN, HW = 8, 96, 1024
K, D_C = 8192, 32
KT = 1024
NKT = K // KT
NTOK = B * HW
D_PAD = 128
BIG = 1e9

_HI = jax.lax.Precision.HIGHEST


def _vq_body(x_ref, cb_ref, win_ref, bin_ref, wout_ref, bout_ref,
             idx_ref, cbout_ref, loss_ref):
    b = pl.program_id(0)

    xb = x_ref[0]
    z = jax.lax.dot_general(xb, win_ref[...],
                            (((0,), (0,)), ((), ())),
                            preferred_element_type=jnp.float32)
    z = z + bin_ref[...]
    z2 = jnp.sum(z * z, axis=1, keepdims=True)

    cols_f = jax.lax.broadcasted_iota(jnp.int32, (HW, KT), 1).astype(jnp.float32)

    best = jnp.full((HW,), BIG, dtype=jnp.float32)
    best_idx = jnp.zeros((HW,), dtype=jnp.float32)
    for j in range(NKT):
        c = cb_ref[pl.ds(j * KT, KT), :]
        s = jax.lax.dot_general(z, c, (((1,), (1,)), ((), ())),
                                preferred_element_type=jnp.float32)
        e2 = jnp.sum(c * c, axis=1)
        dist = (z2 + e2[None, :]) - 2.0 * s
        dmin = jnp.min(dist, axis=1)
        lidx = jnp.min(jnp.where(dist == dmin[:, None], cols_f, BIG),
                       axis=1) + float(j * KT)
        upd = dmin < best
        best = jnp.where(upd, dmin, best)
        best_idx = jnp.where(upd, lidx, best_idx)

    idx_ref[0, 0, :] = best_idx.astype(jnp.int32)

    @pl.when(b == 0)
    def _():
        cbout_ref[...] = jax.lax.dot_general(
            cb_ref[...], wout_ref[...], (((1,), (0,)), ((), ())),
            preferred_element_type=jnp.float32) + bout_ref[...]

    part = jnp.sum(best).reshape(1, 1)

    @pl.when(b == 0)
    def _():
        loss_ref[...] = jnp.zeros((1, 1), jnp.float32)

    loss_ref[...] = loss_ref[...] + part

    @pl.when(b == B - 1)
    def _():
        loss_ref[...] = loss_ref[...] * (1.0 / (NTOK * D_C))


_NC, _NS = 2, 16
_NW = _NC * _NS
BPW = NTOK // _NW
NCHUNK = 4


@functools.cache
def _make_sc_gather():
    mesh = plsc.VectorSubcoreMesh(core_axis_name="c", subcore_axis_name="s")

    @functools.partial(
        pl.kernel,
        mesh=mesh,
        out_type=jax.ShapeDtypeStruct((NTOK, D_PAD), jnp.float32),
        scratch_types=[
            pltpu.VMEM((BPW,), jnp.int32),
            pltpu.VMEM((BPW, D_PAD), jnp.float32),
            pltpu.SemaphoreType.DMA,
        ],
    )
    def _sc_gather(idx_hbm, table_hbm, out_hbm, idx_v, rows_v, sem):
        wid = lax.axis_index("s") * _NC + lax.axis_index("c")
        base = wid * BPW
        pltpu.sync_copy(idx_hbm.at[pl.ds(base, BPW)], idx_v)
        ch = BPW // NCHUNK
        descs = [
            pltpu.async_copy(
                table_hbm.at[idx_v.at[pl.ds(c * ch, ch)]],
                rows_v.at[pl.ds(c * ch, ch)],
                sem,
            )
            for c in range(NCHUNK)
        ]
        for d in descs:
            d.wait()
        pltpu.sync_copy(rows_v, out_hbm.at[pl.ds(base, BPW)])

    return _sc_gather


@functools.partial(jax.jit)
def kernel(x, codebook, Win, b_in, Wout, b_out):
    x3 = x.reshape(B, D_IN, HW)
    idx, cbout, loss = pl.pallas_call(
        _vq_body,
        grid=(B,),
        in_specs=[
            pl.BlockSpec((1, D_IN, HW), lambda b: (b, 0, 0)),
            pl.BlockSpec((K, D_C), lambda b: (0, 0)),
            pl.BlockSpec((D_IN, D_C), lambda b: (0, 0)),
            pl.BlockSpec((1, D_C), lambda b: (0, 0)),
            pl.BlockSpec((D_C, D_PAD), lambda b: (0, 0)),
            pl.BlockSpec((1, D_PAD), lambda b: (0, 0)),
        ],
        out_specs=[
            pl.BlockSpec((1, 1, HW), lambda b: (b, 0, 0)),
            pl.BlockSpec((K, D_PAD), lambda b: (0, 0)),
            pl.BlockSpec((1, 1), lambda b: (0, 0)),
        ],
        out_shape=[
            jax.ShapeDtypeStruct((B, 1, HW), jnp.int32),
            jax.ShapeDtypeStruct((K, D_PAD), jnp.float32),
            jax.ShapeDtypeStruct((1, 1), jnp.float32),
        ],
        compiler_params=pltpu.CompilerParams(
            dimension_semantics=("arbitrary",),
        ),
    )(x3, codebook, Win, b_in.reshape(1, D_C),
      jnp.pad(Wout, ((0, 0), (0, D_PAD - D_IN))),
      jnp.pad(b_out, (0, D_PAD - D_IN)).reshape(1, D_PAD))
    flat_idx = idx.reshape(NTOK)
    gathered = _make_sc_gather()(flat_idx, cbout)
    out = gathered[:, :D_IN].reshape(B, HW, D_IN)
    out = out.transpose(0, 2, 1).reshape(B, D_IN, 32, 32)
    return out, idx.reshape(B, HW), loss.reshape(())


# speedup vs baseline: 1.2810x; 1.2810x over previous
import functools

import jax
import jax.numpy as jnp
from jax import lax
from jax.experimental import pallas as pl
from jax.experimental.pallas import tpu as pltpu
from jax.experimental.pallas import tpu_sc as plsc

B, D_IN, HW = 8, 96, 1024
K, D_C = 8192, 32
KT = 1024
NKT = K // KT
NTOK = B * HW
D_PAD = 128
BIG = 1e9

_HI = jax.lax.Precision.HIGHEST


def _vq_body(x_ref, cb_ref, win_ref, bin_ref, wout_ref, bout_ref,
             idx_ref, cbout_ref, loss_ref):
    b = pl.program_id(0)

    xb = x_ref[0]
    z = jax.lax.dot_general(xb, win_ref[...],
                            (((0,), (0,)), ((), ())),
                            preferred_element_type=jnp.float32)
    z = z + bin_ref[...]
    z2 = jnp.sum(z * z, axis=1, keepdims=True)

    cols_f = jax.lax.broadcasted_iota(jnp.int32, (HW, KT), 1).astype(jnp.float32)

    best = jnp.full((HW,), BIG, dtype=jnp.float32)
    best_idx = jnp.zeros((HW,), dtype=jnp.float32)
    for j in range(NKT):
        c = cb_ref[pl.ds(j * KT, KT), :]
        s = jax.lax.dot_general(z, c, (((1,), (1,)), ((), ())),
                                preferred_element_type=jnp.float32)
        e2 = jnp.sum(c * c, axis=1)
        dist = (z2 + e2[None, :]) - 2.0 * s
        dmin = jnp.min(dist, axis=1)
        lidx = jnp.min(jnp.where(dist == dmin[:, None], cols_f, BIG),
                       axis=1) + float(j * KT)
        upd = dmin < best
        best = jnp.where(upd, dmin, best)
        best_idx = jnp.where(upd, lidx, best_idx)

    idx_ref[0, 0, :] = best_idx.astype(jnp.int32)

    @pl.when(b == 0)
    def _():
        cbout_ref[...] = jax.lax.dot_general(
            cb_ref[...], wout_ref[...], (((1,), (0,)), ((), ())),
            preferred_element_type=jnp.float32) + bout_ref[...]

    part = jnp.sum(best).reshape(1, 1)

    @pl.when(b == 0)
    def _():
        loss_ref[...] = jnp.zeros((1, 1), jnp.float32)

    loss_ref[...] = loss_ref[...] + part

    @pl.when(b == B - 1)
    def _():
        loss_ref[...] = loss_ref[...] * (1.0 / (NTOK * D_C))


_NC, _NS = 2, 16
_NW = _NC * _NS
BPW = NTOK // _NW
NCHUNK = 4


@functools.cache
def _make_sc_gather():
    mesh = plsc.VectorSubcoreMesh(core_axis_name="c", subcore_axis_name="s")

    @functools.partial(
        pl.kernel,
        mesh=mesh,
        out_type=jax.ShapeDtypeStruct((NTOK, D_PAD), jnp.float32),
        scratch_types=[
            pltpu.VMEM((BPW,), jnp.int32),
            pltpu.VMEM((BPW, D_PAD), jnp.float32),
            pltpu.VMEM_SHARED((K, D_PAD), jnp.float32),
            pltpu.SemaphoreType.DMA,
        ],
    )
    def _sc_gather(idx_hbm, table_hbm, out_hbm, idx_v, rows_v, shared, sem):
        sid = lax.axis_index("s")
        wid = sid * _NC + lax.axis_index("c")
        base = wid * BPW
        rps = K // _NS
        pltpu.sync_copy(table_hbm.at[pl.ds(sid * rps, rps)],
                        shared.at[pl.ds(sid * rps, rps)])
        pltpu.sync_copy(idx_hbm.at[pl.ds(base, BPW)], idx_v)
        plsc.subcore_barrier()
        pltpu.async_copy(shared.at[idx_v], rows_v, sem).wait()
        pltpu.sync_copy(rows_v, out_hbm.at[pl.ds(base, BPW)])

    return _sc_gather


@functools.partial(jax.jit)
def kernel(x, codebook, Win, b_in, Wout, b_out):
    x3 = x.reshape(B, D_IN, HW)
    idx, cbout, loss = pl.pallas_call(
        _vq_body,
        grid=(B,),
        in_specs=[
            pl.BlockSpec((1, D_IN, HW), lambda b: (b, 0, 0)),
            pl.BlockSpec((K, D_C), lambda b: (0, 0)),
            pl.BlockSpec((D_IN, D_C), lambda b: (0, 0)),
            pl.BlockSpec((1, D_C), lambda b: (0, 0)),
            pl.BlockSpec((D_C, D_PAD), lambda b: (0, 0)),
            pl.BlockSpec((1, D_PAD), lambda b: (0, 0)),
        ],
        out_specs=[
            pl.BlockSpec((1, 1, HW), lambda b: (b, 0, 0)),
            pl.BlockSpec((K, D_PAD), lambda b: (0, 0)),
            pl.BlockSpec((1, 1), lambda b: (0, 0)),
        ],
        out_shape=[
            jax.ShapeDtypeStruct((B, 1, HW), jnp.int32),
            jax.ShapeDtypeStruct((K, D_PAD), jnp.float32),
            jax.ShapeDtypeStruct((1, 1), jnp.float32),
        ],
        compiler_params=pltpu.CompilerParams(
            dimension_semantics=("arbitrary",),
        ),
    )(x3, codebook, Win, b_in.reshape(1, D_C),
      jnp.pad(Wout, ((0, 0), (0, D_PAD - D_IN))),
      jnp.pad(b_out, (0, D_PAD - D_IN)).reshape(1, D_PAD))
    flat_idx = idx.reshape(NTOK)
    gathered = _make_sc_gather()(flat_idx, cbout)
    out = gathered[:, :D_IN].reshape(B, HW, D_IN)
    out = out.transpose(0, 2, 1).reshape(B, D_IN, 32, 32)
    return out, idx.reshape(B, HW), loss.reshape(())


# speedup vs baseline: 1.2835x; 1.0020x over previous
import functools

import jax
import jax.numpy as jnp
from jax import lax
from jax.experimental import pallas as pl
from jax.experimental.pallas import tpu as pltpu
from jax.experimental.pallas import tpu_sc as plsc

B, D_IN, HW = 8, 96, 1024
K, D_C = 8192, 32
KT = 1024
NKT = K // KT
NTOK = B * HW
D_PAD = 128
BIG = 1e9


def _vq_body(x_ref, cb_ref, win_ref, bin_ref, wout_ref, bout_ref,
             idx_ref, cbout_ref, loss_ref):
    b = pl.program_id(0)

    xb = x_ref[0]
    z = jax.lax.dot_general(xb, win_ref[...],
                            (((0,), (0,)), ((), ())),
                            preferred_element_type=jnp.float32)
    z = z + bin_ref[...]
    z2 = jnp.sum(z * z, axis=1, keepdims=True)

    cols_f = jax.lax.broadcasted_iota(jnp.int32, (HW, KT), 1).astype(jnp.float32)

    best = jnp.full((HW,), BIG, dtype=jnp.float32)
    best_idx = jnp.zeros((HW,), dtype=jnp.float32)
    for j in range(NKT):
        c = cb_ref[pl.ds(j * KT, KT), :]
        s = jax.lax.dot_general(z, c, (((1,), (1,)), ((), ())),
                                preferred_element_type=jnp.float32)
        e2 = jnp.sum(c * c, axis=1)
        dist = (z2 + e2[None, :]) - 2.0 * s
        dmin = jnp.min(dist, axis=1)
        lidx = jnp.min(jnp.where(dist == dmin[:, None], cols_f, BIG),
                       axis=1) + float(j * KT)
        upd = dmin < best
        best = jnp.where(upd, dmin, best)
        best_idx = jnp.where(upd, lidx, best_idx)

    idx_ref[0, 0, :] = best_idx.astype(jnp.int32)

    @pl.when(b == 0)
    def _():
        cbout_ref[...] = jax.lax.dot_general(
            cb_ref[...], wout_ref[...], (((1,), (0,)), ((), ())),
            preferred_element_type=jnp.float32) + bout_ref[...]

    part = jnp.sum(best).reshape(1, 1)

    @pl.when(b == 0)
    def _():
        loss_ref[...] = jnp.zeros((1, 1), jnp.float32)

    loss_ref[...] = loss_ref[...] + part

    @pl.when(b == B - 1)
    def _():
        loss_ref[...] = loss_ref[...] * (1.0 / (NTOK * D_C))


_NC, _NS = 2, 16
_NW = _NC * _NS
BPW = NTOK // _NW


@functools.cache
def _make_sc_gather():
    mesh = plsc.VectorSubcoreMesh(core_axis_name="c", subcore_axis_name="s")

    @functools.partial(
        pl.kernel,
        mesh=mesh,
        out_type=jax.ShapeDtypeStruct((NTOK, D_PAD), jnp.float32),
        scratch_types=[
            pltpu.VMEM((BPW,), jnp.int32),
            pltpu.VMEM((BPW, D_PAD), jnp.float32),
            pltpu.VMEM_SHARED((K, D_PAD), jnp.float32),
            pltpu.SemaphoreType.DMA,
        ],
    )
    def _sc_gather(idx_hbm, table_hbm, out_hbm, idx_v, rows_v, shared, sem):
        sid = lax.axis_index("s")
        wid = sid * _NC + lax.axis_index("c")
        base = wid * BPW
        rps = K // _NS
        pltpu.sync_copy(table_hbm.at[pl.ds(sid * rps, rps)],
                        shared.at[pl.ds(sid * rps, rps)])
        pltpu.sync_copy(idx_hbm.at[pl.ds(base, BPW)], idx_v)
        plsc.subcore_barrier()
        pltpu.async_copy(shared.at[idx_v], rows_v, sem).wait()
        pltpu.sync_copy(rows_v, out_hbm.at[pl.ds(base, BPW)])

    return _sc_gather


@functools.partial(jax.jit)
def kernel(x, codebook, Win, b_in, Wout, b_out):
    x3 = x.reshape(B, D_IN, HW)
    idx, cbout, loss = pl.pallas_call(
        _vq_body,
        grid=(B,),
        in_specs=[
            pl.BlockSpec((1, D_IN, HW), lambda b: (b, 0, 0)),
            pl.BlockSpec((K, D_C), lambda b: (0, 0)),
            pl.BlockSpec((D_IN, D_C), lambda b: (0, 0)),
            pl.BlockSpec((1, D_C), lambda b: (0, 0)),
            pl.BlockSpec((D_C, D_PAD), lambda b: (0, 0)),
            pl.BlockSpec((1, D_PAD), lambda b: (0, 0)),
        ],
        out_specs=[
            pl.BlockSpec((1, 1, HW), lambda b: (b, 0, 0)),
            pl.BlockSpec((K, D_PAD), lambda b: (0, 0)),
            pl.BlockSpec((1, 1), lambda b: (0, 0)),
        ],
        out_shape=[
            jax.ShapeDtypeStruct((B, 1, HW), jnp.int32),
            jax.ShapeDtypeStruct((K, D_PAD), jnp.float32),
            jax.ShapeDtypeStruct((1, 1), jnp.float32),
        ],
        compiler_params=pltpu.CompilerParams(
            dimension_semantics=("arbitrary",),
        ),
    )(x3, codebook, Win, b_in.reshape(1, D_C),
      jnp.pad(Wout, ((0, 0), (0, D_PAD - D_IN))),
      jnp.pad(b_out, (0, D_PAD - D_IN)).reshape(1, D_PAD))
    flat_idx = idx.reshape(NTOK)
    gathered = _make_sc_gather()(flat_idx, cbout)
    out = gathered[:, :D_IN].reshape(B, HW, D_IN)
    out = out.transpose(0, 2, 1).reshape(B, D_IN, 32, 32)
    return out, idx.reshape(B, HW), loss.reshape(())
